# Initial kernel scaffold; baseline (speedup 1.0000x reference)
import jax
import jax.numpy as jnp
from jax.experimental import pallas as pl


def kernel(X, edge_index, W0, as0, ad0, b0, W1, as1, ad1, b1, W2, as2, ad2, b2, W3, as3, ad3, b3, W4, as4, ad4, b4):
    raise NotImplementedError("write your pallas kernel here")


# speedup vs baseline: 1.0078x; 1.0078x over previous
import functools

import jax
import jax.numpy as jnp
from jax.experimental import pallas as pl
from jax.experimental.pallas import tpu as pltpu

N = 10000
E = 320000
H = 2
OC = 200

_BLK = 400


def _proj_body(x_ref, w_ref, asv_ref, adv_ref, h_ref, als_ref, ald_ref, *, heads):
    x = x_ref[...]
    w = w_ref[...]
    h = jnp.dot(x, w, preferred_element_type=jnp.float32)
    h_ref[...] = h
    hv = h.reshape(x.shape[0], heads, OC)
    als_ref[...] = (hv * asv_ref[...][None]).sum(-1)
    ald_ref[...] = (hv * adv_ref[...][None]).sum(-1)


def _project(x, w, a_src, a_dst, heads):
    din = x.shape[1]
    dout = heads * OC
    grid = (N // _BLK,)
    return pl.pallas_call(
        functools.partial(_proj_body, heads=heads),
        grid=grid,
        in_specs=[
            pl.BlockSpec((_BLK, din), lambda i: (i, 0)),
            pl.BlockSpec((din, dout), lambda i: (0, 0)),
            pl.BlockSpec((heads, OC), lambda i: (0, 0)),
            pl.BlockSpec((heads, OC), lambda i: (0, 0)),
        ],
        out_specs=[
            pl.BlockSpec((_BLK, dout), lambda i: (i, 0)),
            pl.BlockSpec((_BLK, heads), lambda i: (i, 0)),
            pl.BlockSpec((_BLK, heads), lambda i: (i, 0)),
        ],
        out_shape=[
            jax.ShapeDtypeStruct((N, dout), jnp.float32),
            jax.ShapeDtypeStruct((N, heads), jnp.float32),
            jax.ShapeDtypeStruct((N, heads), jnp.float32),
        ],
    )(x, w, a_src, a_dst)


def _edge_phase(h, al_s, al_d, src, dst, heads):
    n = h.shape[0]
    alpha = al_s[src] + al_d[dst]
    alpha = jax.nn.leaky_relu(alpha, 0.2)
    amax = jax.ops.segment_max(alpha, dst, num_segments=n)
    ex = jnp.exp(alpha - amax[dst])
    den = jax.ops.segment_sum(ex, dst, num_segments=n)
    att = ex / (den[dst] + 1e-16)
    msg = h.reshape(n, heads, OC)[src] * att[:, :, None]
    out = jax.ops.segment_sum(msg, dst, num_segments=n).reshape(n, heads * OC)
    return out, att


def kernel(X, edge_index, W0, as0, ad0, b0, W1, as1, ad1, b1, W2, as2, ad2, b2,
           W3, as3, ad3, b3, W4, as4, ad4, b4):
    n = X.shape[0]
    loop = jnp.arange(n, dtype=edge_index.dtype)
    src = jnp.concatenate([edge_index[0], loop])
    dst = jnp.concatenate([edge_index[1], loop])
    act = lambda t: jax.nn.gelu(t, approximate=False)

    x = X
    layers = [(W0, as0, ad0, b0, H), (W1, as1, ad1, b1, H), (W2, as2, ad2, b2, H),
              (W3, as3, ad3, b3, H), (W4, as4, ad4, b4, 1)]
    att = None
    for (w, a_s, a_d, b, heads) in layers:
        h, al_s, al_d = _project(x, w, a_s, a_d, heads)
        out, att = _edge_phase(h, al_s, al_d, src, dst, heads)
        x = act(out + b)
    return x, att


# speedup vs baseline: 12.4053x; 12.3096x over previous
import functools

import jax
import jax.numpy as jnp
from jax import lax
from jax.experimental import pallas as pl
from jax.experimental.pallas import tpu as pltpu
from jax.experimental.pallas import tpu_sc as plsc

N = 10000
NP = 10240
E = 320000
E2 = E + N
OC = 200
SW = 100
DP = 128

NT = 16
BE = 32
G = 8
NBG = 84
LS = NBG * BE
CH = G * LS
E2P = CH * NT
NEG = -1e30
EPS = 1e-16

_BLK = 400


def _gelu(x):
    return 0.5 * x * (1.0 + lax.erf(x * jnp.float32(0.7071067811865476)))


def _emit_proj(h, asv_ref, adv_ref, hp_ref, als_ref, ald_ref, heads):
    b = h.shape[0]
    z = jnp.zeros((b, DP - SW), jnp.float32)
    hp = jnp.stack([jnp.concatenate([h[:, s * SW:(s + 1) * SW], z], 1)
                    for s in range(2 * heads)], axis=0)
    if heads == 2:
        h0, h1 = h[:, :OC], h[:, OC:]
        al_s = jnp.stack([(h0 * asv_ref[0][None]).sum(-1),
                          (h1 * asv_ref[1][None]).sum(-1)], axis=0)
        al_d = jnp.stack([(h0 * adv_ref[0][None]).sum(-1),
                          (h1 * adv_ref[1][None]).sum(-1)], axis=0)
    else:
        a_s = (h * asv_ref[0][None]).sum(-1)
        a_d = (h * adv_ref[0][None]).sum(-1)
        al_s = jnp.stack([a_s, a_s], axis=0)
        al_d = jnp.stack([a_d, a_d], axis=0)
    hp_ref[...] = hp
    als_ref[...] = al_s[None]
    ald_ref[...] = al_d[None]


def _proj_body(x_ref, w_ref, asv_ref, adv_ref, hp_ref, als_ref, ald_ref,
               *, heads):
    h = jnp.dot(x_ref[...], w_ref[...], preferred_element_type=jnp.float32)
    _emit_proj(h, asv_ref, adv_ref, hp_ref, als_ref, ald_ref, heads)


def _proj2_body(acc_ref, den_ref, b_ref, w_ref, asv_ref, adv_ref,
                hp_ref, als_ref, ald_ref, *, heads):
    acc = acc_ref[...]
    den = den_ref[0]
    bv = b_ref[...]
    xs = []
    for s in range(4):
        xs.append(acc[s, :, :SW] / (den[s // 2][:, None] + EPS) +
                  bv[0, s * SW:(s + 1) * SW][None])
    x = _gelu(jnp.concatenate(xs, axis=1))
    h = jnp.dot(x, w_ref[...], preferred_element_type=jnp.float32)
    _emit_proj(h, asv_ref, adv_ref, hp_ref, als_ref, ald_ref, heads)


def _project(x_or_acc, den, bias, w, a_src, a_dst, heads):
    dout = heads * OC
    grid = (N // _BLK,)
    nb = N // _BLK
    ns = 2 * heads
    out_specs = [
        pl.BlockSpec((ns, _BLK, DP), lambda i: (0, i, 0)),
        pl.BlockSpec((1, 2, _BLK), lambda i: (i, 0, 0)),
        pl.BlockSpec((1, 2, _BLK), lambda i: (i, 0, 0)),
    ]
    out_shape = [
        jax.ShapeDtypeStruct((ns, N, DP), jnp.float32),
        jax.ShapeDtypeStruct((nb, 2, _BLK), jnp.float32),
        jax.ShapeDtypeStruct((nb, 2, _BLK), jnp.float32),
    ]
    ah = a_src.shape[0]
    if den is None:
        din = x_or_acc.shape[1]
        return pl.pallas_call(
            functools.partial(_proj_body, heads=heads),
            grid=grid,
            in_specs=[
                pl.BlockSpec((_BLK, din), lambda i: (i, 0)),
                pl.BlockSpec((din, dout), lambda i: (0, 0)),
                pl.BlockSpec((ah, OC), lambda i: (0, 0)),
                pl.BlockSpec((ah, OC), lambda i: (0, 0)),
            ],
            out_specs=out_specs,
            out_shape=out_shape,
        )(x_or_acc, w, a_src, a_dst)
    din = 2 * OC
    return pl.pallas_call(
        functools.partial(_proj2_body, heads=heads),
        grid=grid,
        in_specs=[
            pl.BlockSpec((4, _BLK, DP), lambda i: (0, i, 0)),
            pl.BlockSpec((1, 2, _BLK), lambda i: (i, 0, 0)),
            pl.BlockSpec((1, din), lambda i: (0, 0)),
            pl.BlockSpec((din, dout), lambda i: (0, 0)),
            pl.BlockSpec((ah, OC), lambda i: (0, 0)),
            pl.BlockSpec((ah, OC), lambda i: (0, 0)),
        ],
        out_specs=out_specs,
        out_shape=out_shape,
    )(x_or_acc, den, bias.reshape(1, din), w, a_src, a_dst)


def _final_body(acc_ref, den_ref, b_ref, out_ref):
    acc = acc_ref[...]
    den = den_ref[0]
    x0 = acc[0, :, :SW] / (den[0][:, None] + EPS)
    x1 = acc[1, :, :SW] / (den[1][:, None] + EPS)
    x = jnp.concatenate([x0, x1], axis=1) + b_ref[...][0][None]
    out_ref[...] = _gelu(x)


def _finalize(acc, den, bias):
    return pl.pallas_call(
        _final_body,
        grid=(N // _BLK,),
        in_specs=[
            pl.BlockSpec((2, _BLK, DP), lambda i: (0, i, 0)),
            pl.BlockSpec((1, 2, _BLK), lambda i: (i, 0, 0)),
            pl.BlockSpec((1, OC), lambda i: (0, 0)),
        ],
        out_specs=pl.BlockSpec((_BLK, OC), lambda i: (i, 0)),
        out_shape=jax.ShapeDtypeStruct((N, OC), jnp.float32),
    )(acc, den, bias.reshape(1, OC))


def _edge_impl(src_hbm, dst_hbm, als_hbm, ald_hbm, hp_hbm,
               acc_hbm, den_hbm, att_hbm,
               src_v, dst_v, att_v, als_v, ald_v, den_v,
               rowbuf, cmax_v, ebuf_v, sbuf_v, dbuf_v,
               den_sp, acc_sp, cst_sp, sem,
               *, want_att):
    slot = lax.axis_index("c")
    tid = lax.axis_index("s")
    base = tid * jnp.int32(CH)
    limit = jnp.minimum(jnp.int32(CH), jnp.int32(E2) - base)
    iota = lax.iota(jnp.int32, 16)

    pltpu.sync_copy(als_hbm.at[pl.ds(slot * NP, NP)], als_v)
    pltpu.sync_copy(ald_hbm.at[pl.ds(slot * NP, NP)], ald_v)

    def stage(g):
        pltpu.sync_copy(src_hbm.at[pl.ds(base + g * LS, LS)], src_v)
        pltpu.sync_copy(dst_hbm.at[pl.ds(base + g * LS, LS)], dst_v)

    def alpha_block(g, i, k):
        s16 = src_v[pl.ds(i * BE + k * 16, 16)]
        d16 = dst_v[pl.ds(i * BE + k * 16, 16)]
        a = (plsc.load_gather(als_v, [s16]) +
             plsc.load_gather(ald_v, [d16]))
        a = jnp.where(a >= 0.0, a, 0.2 * a)
        loc = jnp.full((16,), g * LS + i * BE + k * 16, jnp.int32) + iota
        a = jnp.where(loc < limit, a, jnp.float32(NEG))
        return a, d16, s16

    def strip_max(g, m):
        stage(g)

        def blk(i, m):
            for k in range(BE // 16):
                a, _, _ = alpha_block(g, i, k)
                m = jnp.maximum(m, a)
            return m

        return lax.fori_loop(0, NBG, blk, m)

    m = lax.fori_loop(0, G, strip_max, jnp.full((16,), NEG, jnp.float32))
    lmv = jnp.broadcast_to(jnp.max(m), (16,))
    for r in range(16):
        cmax_v[r, pl.ds(0, 16)] = lmv
    pltpu.sync_copy(cmax_v, cst_sp.at[jnp.full((16,), tid, jnp.int32)])
    plsc.subcore_barrier()
    pltpu.sync_copy(cst_sp, cmax_v)
    mm = jnp.full((16,), NEG, jnp.float32)
    for r in range(16):
        mm = jnp.maximum(mm, cmax_v[r, pl.ds(0, 16)])
    cvec = jnp.broadcast_to(jnp.max(mm), (16,))

    def zden(j, _):
        for j2 in range(8):
            den_v[j, pl.ds(j2 * 16, 16)] = jnp.zeros((16,), jnp.float32)
        return 0

    lax.fori_loop(0, 80, zden, 0)

    def strip_den(g, _):
        stage(g)

        def blk(i, _):
            for k in range(BE // 16):
                a, d16, _ = alpha_block(g, i, k)
                e = jnp.exp(a - cvec)
                plsc.addupdate_scatter(
                    den_v,
                    [jnp.right_shift(d16, 7), jnp.bitwise_and(d16, 127)],
                    e)
            return 0

        return lax.fori_loop(0, NBG, blk, 0)

    lax.fori_loop(0, G, strip_den, 0)

    @pl.when(tid == 0)
    def _():
        pltpu.sync_copy(den_v, den_sp)

    plsc.subcore_barrier()

    @pl.when(tid != 0)
    def _():
        def dadd(c, _):
            pltpu.sync_copy(den_v.at[pl.ds(c * 16, 16)],
                            den_sp.at[c * 16 + iota], add=True)
            return 0
        lax.fori_loop(0, 5, dadd, 0)

    plsc.subcore_barrier()
    pltpu.sync_copy(den_sp, den_v)

    @pl.when(jnp.logical_and(tid == 0, slot == 0))
    def _():
        pltpu.sync_copy(den_v, den_hbm)

    for r in range(16):
        for j in range(DP // 16):
            rowbuf[r, pl.ds(j * 16, 16)] = jnp.zeros((16,), jnp.float32)

    def zacc(k, _):
        pltpu.sync_copy(rowbuf.at[pl.ds(0, 16)],
                        acc_sp.at[pl.ds(tid * 624 + k * 16, 16)])
        return 0

    lax.fori_loop(0, 39, zacc, 0)

    @pl.when(tid == 0)
    def _():
        pltpu.sync_copy(rowbuf.at[pl.ds(0, 16)],
                        acc_sp.at[pl.ds(9984, 16)])

    plsc.subcore_barrier()

    soff = slot * jnp.int32(N)

    def strip_msg(g, _):
        stage(g)

        def blk(i, _):
            for k in range(BE // 16):
                a, d16, s16 = alpha_block(g, i, k)
                sbuf_v[pl.ds(k * 16, 16)] = s16 + soff
                dbuf_v[pl.ds(k * 16, 16)] = d16
                e = jnp.exp(a - cvec)
                ebuf_v[pl.ds(k * 16, 16)] = e
                if want_att:
                    dg = plsc.load_gather(
                        den_v, [jnp.right_shift(d16, 7),
                                jnp.bitwise_and(d16, 127)])
                    att_v[pl.ds(i * BE + k * 16, 16)] = (
                        e / (dg + jnp.float32(EPS)))
            pltpu.async_copy(hp_hbm.at[sbuf_v], rowbuf, sem).wait()
            for row in range(BE):
                ws = plsc.load_gather(
                    ebuf_v, [jnp.full((16,), row, jnp.int32)])
                for j in range(DP // 16):
                    rowbuf[row, pl.ds(j * 16, 16)] = (
                        rowbuf[row, pl.ds(j * 16, 16)] * ws)
            pltpu.sync_copy(rowbuf, acc_sp.at[dbuf_v], add=True)
            return 0

        lax.fori_loop(0, NBG, blk, 0)
        if want_att:
            @pl.when(slot == 0)
            def _():
                pltpu.sync_copy(att_v,
                                att_hbm.at[pl.ds(base + g * LS, LS)])
        return 0

    lax.fori_loop(0, G, strip_msg, 0)
    plsc.subcore_barrier()

    def wb_chunk(r0):
        pltpu.sync_copy(acc_sp.at[pl.ds(r0, 16)], rowbuf.at[pl.ds(0, 16)])
        pltpu.sync_copy(rowbuf.at[pl.ds(0, 16)],
                        acc_hbm.at[slot, pl.ds(r0, 16)])

    def wb(k, _):
        wb_chunk(tid * 624 + k * 16)
        return 0

    lax.fori_loop(0, 39, wb, 0)

    @pl.when(tid == 0)
    def _():
        wb_chunk(jnp.int32(9984))


@functools.lru_cache(maxsize=None)
def _edge_kernel(want_att):
    out_type = [
        jax.ShapeDtypeStruct((2, N, DP), jnp.float32),
        jax.ShapeDtypeStruct((80, 128), jnp.float32),
    ]
    if want_att:
        out_type.append(jax.ShapeDtypeStruct((E2P,), jnp.float32))
    scratch = [
        pltpu.VMEM((LS,), jnp.int32),
        pltpu.VMEM((LS,), jnp.int32),
    ]
    if want_att:
        scratch.append(pltpu.VMEM((LS,), jnp.float32))
    scratch += [
        pltpu.VMEM((NP,), jnp.float32),
        pltpu.VMEM((NP,), jnp.float32),
        pltpu.VMEM((80, 128), jnp.float32),
        pltpu.VMEM((BE, DP), jnp.float32),
        pltpu.VMEM((16, 16), jnp.float32),
        pltpu.VMEM((BE,), jnp.float32),
        pltpu.VMEM((BE,), jnp.int32),
        pltpu.VMEM((BE,), jnp.int32),
        pltpu.VMEM_SHARED((80, 128), jnp.float32),
        pltpu.VMEM_SHARED((N, DP), jnp.float32),
        pltpu.VMEM_SHARED((16, 16), jnp.float32),
        pltpu.SemaphoreType.DMA,
    ]

    if want_att:
        def body(src, dst, als, ald, hp, acc, den, att,
                 src_v, dst_v, att_v, als_v, ald_v, den_v,
                 rowbuf, cmax_v, ebuf_v, sbuf_v, dbuf_v,
                 den_sp, acc_sp, cst_sp, sem):
            _edge_impl(src, dst, als, ald, hp, acc, den, att,
                       src_v, dst_v, att_v, als_v, ald_v, den_v,
                       rowbuf, cmax_v, ebuf_v, sbuf_v, dbuf_v,
                       den_sp, acc_sp, cst_sp, sem, want_att=True)
    else:
        def body(src, dst, als, ald, hp, acc, den,
                 src_v, dst_v, als_v, ald_v, den_v,
                 rowbuf, cmax_v, ebuf_v, sbuf_v, dbuf_v,
                 den_sp, acc_sp, cst_sp, sem):
            _edge_impl(src, dst, als, ald, hp, acc, den, None,
                       src_v, dst_v, None, als_v, ald_v, den_v,
                       rowbuf, cmax_v, ebuf_v, sbuf_v, dbuf_v,
                       den_sp, acc_sp, cst_sp, sem, want_att=False)

    mesh = plsc.VectorSubcoreMesh(core_axis_name="c", subcore_axis_name="s")
    return pl.kernel(
        body, out_type=out_type, mesh=mesh, scratch_types=scratch,
        compiler_params=pltpu.CompilerParams(needs_layout_passes=False))


def kernel(X, edge_index, W0, as0, ad0, b0, W1, as1, ad1, b1, W2, as2, ad2,
           b2, W3, as3, ad3, b3, W4, as4, ad4, b4):
    ei = edge_index.astype(jnp.int32)
    loop = jnp.arange(N, dtype=jnp.int32)
    src = jnp.concatenate([ei[0], loop])
    dst = jnp.concatenate([ei[1], loop])
    src1 = jnp.pad(src, (0, E2P - E2))
    dst1 = jnp.pad(dst, (0, E2P - E2))

    layers = [(W0, as0, ad0, b0, 2), (W1, as1, ad1, b1, 2),
              (W2, as2, ad2, b2, 2), (W3, as3, ad3, b3, 2),
              (W4, as4, ad4, b4, 1)]

    nb = N // _BLK
    acc = den_r = None
    bias_prev = None
    att = None
    for li, (w, a_s, a_d, b, heads) in enumerate(layers):
        if li == 0:
            hp, als, ald = _project(X, None, None, w, a_s, a_d, heads)
        else:
            hp, als, ald = _project(acc, den_r, bias_prev, w, a_s, a_d,
                                    heads)
        als_t = als.transpose(1, 0, 2).reshape(2, N)
        ald_t = ald.transpose(1, 0, 2).reshape(2, N)
        accs = []
        dens = []
        for hd in range(heads):
            alh = jnp.pad(als_t[hd], (0, NP - N))
            adh = jnp.pad(ald_t[hd], (0, NP - N))
            als2 = jnp.concatenate([alh, alh])
            ald2 = jnp.concatenate([adh, adh])
            hp2 = hp[2 * hd:2 * hd + 2].reshape(2 * N, DP)
            ek = _edge_kernel(li == 4)
            outs = ek(src1, dst1, als2, ald2, hp2)
            if li == 4:
                acc_h, den_h, attp = outs
                att = attp[:E2][:, None]
            else:
                acc_h, den_h = outs
            accs.append(acc_h)
            dens.append(den_h.reshape(80 * 128)[:N])
        acc = jnp.concatenate(accs, axis=0)
        if heads == 1:
            dens = [dens[0], dens[0]]
        den_r = (jnp.stack(dens).reshape(2, nb, _BLK).transpose(1, 0, 2))
        bias_prev = b

    x = _finalize(acc, den_r, b4)
    return x, att


# speedup vs baseline: 13.7716x; 1.1101x over previous
import functools

import jax
import jax.numpy as jnp
from jax import lax
from jax.experimental import pallas as pl
from jax.experimental.pallas import tpu as pltpu
from jax.experimental.pallas import tpu_sc as plsc

N = 10000
NP = 10240
E = 320000
E2 = E + N
OC = 200
SW = 100
DP = 128

NT = 16
BE = 64
G = 8
NBG = 42
LS = NBG * BE
CH = G * LS
E2P = CH * NT
NEG = -1e30
EPS = 1e-16

_BLK = 400


def _gelu(x):
    return 0.5 * x * (1.0 + lax.erf(x * jnp.float32(0.7071067811865476)))


def _emit_proj(h, asv_ref, adv_ref, hp_ref, als_ref, ald_ref, heads):
    b = h.shape[0]
    z = jnp.zeros((b, DP - SW), jnp.float32)
    hp = jnp.stack([jnp.concatenate([h[:, s * SW:(s + 1) * SW], z], 1)
                    for s in range(2 * heads)], axis=0)
    if heads == 2:
        h0, h1 = h[:, :OC], h[:, OC:]
        al_s = jnp.stack([(h0 * asv_ref[0][None]).sum(-1),
                          (h1 * asv_ref[1][None]).sum(-1)], axis=0)
        al_d = jnp.stack([(h0 * adv_ref[0][None]).sum(-1),
                          (h1 * adv_ref[1][None]).sum(-1)], axis=0)
    else:
        a_s = (h * asv_ref[0][None]).sum(-1)
        a_d = (h * adv_ref[0][None]).sum(-1)
        al_s = jnp.stack([a_s, a_s], axis=0)
        al_d = jnp.stack([a_d, a_d], axis=0)
    hp_ref[...] = hp
    als_ref[...] = al_s[None]
    ald_ref[...] = al_d[None]


def _proj_body(x_ref, w_ref, asv_ref, adv_ref, hp_ref, als_ref, ald_ref,
               *, heads):
    h = jnp.dot(x_ref[...], w_ref[...], preferred_element_type=jnp.float32)
    _emit_proj(h, asv_ref, adv_ref, hp_ref, als_ref, ald_ref, heads)


def _proj2_body(acc_ref, den_ref, b_ref, w_ref, asv_ref, adv_ref,
                hp_ref, als_ref, ald_ref, *, heads):
    acc = acc_ref[...]
    den = den_ref[0]
    bv = b_ref[...]
    xs = []
    for s in range(4):
        xs.append(acc[s, :, :SW] / (den[s // 2][:, None] + EPS) +
                  bv[0, s * SW:(s + 1) * SW][None])
    x = _gelu(jnp.concatenate(xs, axis=1))
    h = jnp.dot(x, w_ref[...], preferred_element_type=jnp.float32)
    _emit_proj(h, asv_ref, adv_ref, hp_ref, als_ref, ald_ref, heads)


def _project(x_or_acc, den, bias, w, a_src, a_dst, heads):
    dout = heads * OC
    grid = (N // _BLK,)
    nb = N // _BLK
    ns = 2 * heads
    out_specs = [
        pl.BlockSpec((ns, _BLK, DP), lambda i: (0, i, 0)),
        pl.BlockSpec((1, 2, _BLK), lambda i: (i, 0, 0)),
        pl.BlockSpec((1, 2, _BLK), lambda i: (i, 0, 0)),
    ]
    out_shape = [
        jax.ShapeDtypeStruct((ns, N, DP), jnp.float32),
        jax.ShapeDtypeStruct((nb, 2, _BLK), jnp.float32),
        jax.ShapeDtypeStruct((nb, 2, _BLK), jnp.float32),
    ]
    ah = a_src.shape[0]
    if den is None:
        din = x_or_acc.shape[1]
        return pl.pallas_call(
            functools.partial(_proj_body, heads=heads),
            grid=grid,
            in_specs=[
                pl.BlockSpec((_BLK, din), lambda i: (i, 0)),
                pl.BlockSpec((din, dout), lambda i: (0, 0)),
                pl.BlockSpec((ah, OC), lambda i: (0, 0)),
                pl.BlockSpec((ah, OC), lambda i: (0, 0)),
            ],
            out_specs=out_specs,
            out_shape=out_shape,
        )(x_or_acc, w, a_src, a_dst)
    din = 2 * OC
    return pl.pallas_call(
        functools.partial(_proj2_body, heads=heads),
        grid=grid,
        in_specs=[
            pl.BlockSpec((4, _BLK, DP), lambda i: (0, i, 0)),
            pl.BlockSpec((1, 2, _BLK), lambda i: (i, 0, 0)),
            pl.BlockSpec((1, din), lambda i: (0, 0)),
            pl.BlockSpec((din, dout), lambda i: (0, 0)),
            pl.BlockSpec((ah, OC), lambda i: (0, 0)),
            pl.BlockSpec((ah, OC), lambda i: (0, 0)),
        ],
        out_specs=out_specs,
        out_shape=out_shape,
    )(x_or_acc, den, bias.reshape(1, din), w, a_src, a_dst)


def _final_body(acc_ref, den_ref, b_ref, out_ref):
    acc = acc_ref[...]
    den = den_ref[0]
    x0 = acc[0, :, :SW] / (den[0][:, None] + EPS)
    x1 = acc[1, :, :SW] / (den[1][:, None] + EPS)
    x = jnp.concatenate([x0, x1], axis=1) + b_ref[...][0][None]
    out_ref[...] = _gelu(x)


def _finalize(acc, den, bias):
    return pl.pallas_call(
        _final_body,
        grid=(N // _BLK,),
        in_specs=[
            pl.BlockSpec((2, _BLK, DP), lambda i: (0, i, 0)),
            pl.BlockSpec((1, 2, _BLK), lambda i: (i, 0, 0)),
            pl.BlockSpec((1, OC), lambda i: (0, 0)),
        ],
        out_specs=pl.BlockSpec((_BLK, OC), lambda i: (i, 0)),
        out_shape=jax.ShapeDtypeStruct((N, OC), jnp.float32),
    )(acc, den, bias.reshape(1, OC))


def _edge_impl(src_hbm, dst_hbm, als_hbm, ald_hbm, hp_hbm,
               acc_hbm, den_hbm, att_hbm,
               src_v, dst_v, att_v, als_v, ald_v, den_v,
               rowbuf, cmax_v, ebuf_v, sbuf_v, dbuf_v,
               den_sp, acc_sp, cst_sp, sem,
               *, want_att):
    slot = lax.axis_index("c")
    tid = lax.axis_index("s")
    base = tid * jnp.int32(CH)
    limit = jnp.minimum(jnp.int32(CH), jnp.int32(E2) - base)
    iota = lax.iota(jnp.int32, 16)

    pltpu.sync_copy(als_hbm.at[pl.ds(slot * NP, NP)], als_v)
    pltpu.sync_copy(ald_hbm.at[pl.ds(slot * NP, NP)], ald_v)

    def stage(g):
        pltpu.sync_copy(src_hbm.at[pl.ds(base + g * LS, LS)], src_v)
        pltpu.sync_copy(dst_hbm.at[pl.ds(base + g * LS, LS)], dst_v)

    def alpha_block(g, i, k):
        s16 = src_v[pl.ds(i * BE + k * 16, 16)]
        d16 = dst_v[pl.ds(i * BE + k * 16, 16)]
        a = (plsc.load_gather(als_v, [s16]) +
             plsc.load_gather(ald_v, [d16]))
        a = jnp.where(a >= 0.0, a, 0.2 * a)
        loc = jnp.full((16,), g * LS + i * BE + k * 16, jnp.int32) + iota
        a = jnp.where(loc < limit, a, jnp.float32(NEG))
        return a, d16, s16

    def strip_max(g, m):
        stage(g)

        def blk(i, m):
            for k in range(BE // 16):
                a, _, _ = alpha_block(g, i, k)
                m = jnp.maximum(m, a)
            return m

        return lax.fori_loop(0, NBG, blk, m)

    m = lax.fori_loop(0, G, strip_max, jnp.full((16,), NEG, jnp.float32))
    lmv = jnp.broadcast_to(jnp.max(m), (16,))
    for r in range(16):
        cmax_v[r, pl.ds(0, 16)] = lmv
    pltpu.sync_copy(cmax_v, cst_sp.at[jnp.full((16,), tid, jnp.int32)])
    plsc.subcore_barrier()
    pltpu.sync_copy(cst_sp, cmax_v)
    mm = jnp.full((16,), NEG, jnp.float32)
    for r in range(16):
        mm = jnp.maximum(mm, cmax_v[r, pl.ds(0, 16)])
    cvec = jnp.broadcast_to(jnp.max(mm), (16,))

    def zden(j, _):
        for j2 in range(8):
            den_v[j, pl.ds(j2 * 16, 16)] = jnp.zeros((16,), jnp.float32)
        return 0

    lax.fori_loop(0, 80, zden, 0)

    def strip_den(g, _):
        stage(g)

        def blk(i, _):
            for k in range(BE // 16):
                a, d16, _ = alpha_block(g, i, k)
                e = jnp.exp(a - cvec)
                plsc.addupdate_scatter(
                    den_v,
                    [jnp.right_shift(d16, 7), jnp.bitwise_and(d16, 127)],
                    e)
            return 0

        return lax.fori_loop(0, NBG, blk, 0)

    lax.fori_loop(0, G, strip_den, 0)

    @pl.when(tid == 0)
    def _():
        pltpu.sync_copy(den_v, den_sp)

    plsc.subcore_barrier()

    @pl.when(tid != 0)
    def _():
        def dadd(c, _):
            pltpu.sync_copy(den_v.at[pl.ds(c * 16, 16)],
                            den_sp.at[c * 16 + iota], add=True)
            return 0
        lax.fori_loop(0, 5, dadd, 0)

    plsc.subcore_barrier()
    pltpu.sync_copy(den_sp, den_v)

    @pl.when(jnp.logical_and(tid == 0, slot == 0))
    def _():
        pltpu.sync_copy(den_v, den_hbm)

    for r in range(16):
        for j in range(DP // 16):
            rowbuf[r, pl.ds(j * 16, 16)] = jnp.zeros((16,), jnp.float32)

    def zacc(k, _):
        pltpu.sync_copy(rowbuf.at[pl.ds(0, 16)],
                        acc_sp.at[pl.ds(tid * 624 + k * 16, 16)])
        return 0

    lax.fori_loop(0, 39, zacc, 0)

    @pl.when(tid == 0)
    def _():
        pltpu.sync_copy(rowbuf.at[pl.ds(0, 16)],
                        acc_sp.at[pl.ds(9984, 16)])

    plsc.subcore_barrier()

    soff = slot * jnp.int32(N)

    def strip_msg(g, _):
        stage(g)

        def blk(i, _):
            for k in range(BE // 16):
                a, d16, s16 = alpha_block(g, i, k)
                sbuf_v[pl.ds(k * 16, 16)] = s16 + soff
                dbuf_v[pl.ds(k * 16, 16)] = d16
                e = jnp.exp(a - cvec)
                ebuf_v[pl.ds(k * 16, 16)] = e
                if want_att:
                    dg = plsc.load_gather(
                        den_v, [jnp.right_shift(d16, 7),
                                jnp.bitwise_and(d16, 127)])
                    att_v[pl.ds(i * BE + k * 16, 16)] = (
                        e / (dg + jnp.float32(EPS)))
            pltpu.async_copy(hp_hbm.at[sbuf_v], rowbuf, sem).wait()
            for row in range(BE):
                ws = plsc.load_gather(
                    ebuf_v, [jnp.full((16,), row, jnp.int32)])
                for j in range(DP // 16):
                    rowbuf[row, pl.ds(j * 16, 16)] = (
                        rowbuf[row, pl.ds(j * 16, 16)] * ws)
            pltpu.sync_copy(rowbuf, acc_sp.at[dbuf_v], add=True)
            return 0

        lax.fori_loop(0, NBG, blk, 0)
        if want_att:
            @pl.when(slot == 0)
            def _():
                pltpu.sync_copy(att_v,
                                att_hbm.at[pl.ds(base + g * LS, LS)])
        return 0

    lax.fori_loop(0, G, strip_msg, 0)
    plsc.subcore_barrier()

    def wb_chunk(r0):
        pltpu.sync_copy(acc_sp.at[pl.ds(r0, 16)], rowbuf.at[pl.ds(0, 16)])
        pltpu.sync_copy(rowbuf.at[pl.ds(0, 16)],
                        acc_hbm.at[slot, pl.ds(r0, 16)])

    def wb(k, _):
        wb_chunk(tid * 624 + k * 16)
        return 0

    lax.fori_loop(0, 39, wb, 0)

    @pl.when(tid == 0)
    def _():
        wb_chunk(jnp.int32(9984))


@functools.lru_cache(maxsize=None)
def _edge_kernel(want_att):
    out_type = [
        jax.ShapeDtypeStruct((2, N, DP), jnp.float32),
        jax.ShapeDtypeStruct((80, 128), jnp.float32),
    ]
    if want_att:
        out_type.append(jax.ShapeDtypeStruct((E2P,), jnp.float32))
    scratch = [
        pltpu.VMEM((LS,), jnp.int32),
        pltpu.VMEM((LS,), jnp.int32),
    ]
    if want_att:
        scratch.append(pltpu.VMEM((LS,), jnp.float32))
    scratch += [
        pltpu.VMEM((NP,), jnp.float32),
        pltpu.VMEM((NP,), jnp.float32),
        pltpu.VMEM((80, 128), jnp.float32),
        pltpu.VMEM((BE, DP), jnp.float32),
        pltpu.VMEM((16, 16), jnp.float32),
        pltpu.VMEM((BE,), jnp.float32),
        pltpu.VMEM((BE,), jnp.int32),
        pltpu.VMEM((BE,), jnp.int32),
        pltpu.VMEM_SHARED((80, 128), jnp.float32),
        pltpu.VMEM_SHARED((N, DP), jnp.float32),
        pltpu.VMEM_SHARED((16, 16), jnp.float32),
        pltpu.SemaphoreType.DMA,
    ]

    if want_att:
        def body(src, dst, als, ald, hp, acc, den, att,
                 src_v, dst_v, att_v, als_v, ald_v, den_v,
                 rowbuf, cmax_v, ebuf_v, sbuf_v, dbuf_v,
                 den_sp, acc_sp, cst_sp, sem):
            _edge_impl(src, dst, als, ald, hp, acc, den, att,
                       src_v, dst_v, att_v, als_v, ald_v, den_v,
                       rowbuf, cmax_v, ebuf_v, sbuf_v, dbuf_v,
                       den_sp, acc_sp, cst_sp, sem, want_att=True)
    else:
        def body(src, dst, als, ald, hp, acc, den,
                 src_v, dst_v, als_v, ald_v, den_v,
                 rowbuf, cmax_v, ebuf_v, sbuf_v, dbuf_v,
                 den_sp, acc_sp, cst_sp, sem):
            _edge_impl(src, dst, als, ald, hp, acc, den, None,
                       src_v, dst_v, None, als_v, ald_v, den_v,
                       rowbuf, cmax_v, ebuf_v, sbuf_v, dbuf_v,
                       den_sp, acc_sp, cst_sp, sem, want_att=False)

    mesh = plsc.VectorSubcoreMesh(core_axis_name="c", subcore_axis_name="s")
    return pl.kernel(
        body, out_type=out_type, mesh=mesh, scratch_types=scratch,
        compiler_params=pltpu.CompilerParams(needs_layout_passes=False))


def kernel(X, edge_index, W0, as0, ad0, b0, W1, as1, ad1, b1, W2, as2, ad2,
           b2, W3, as3, ad3, b3, W4, as4, ad4, b4):
    ei = edge_index.astype(jnp.int32)
    loop = jnp.arange(N, dtype=jnp.int32)
    src = jnp.concatenate([ei[0], loop])
    dst = jnp.concatenate([ei[1], loop])
    src1 = jnp.pad(src, (0, E2P - E2))
    dst1 = jnp.pad(dst, (0, E2P - E2))

    layers = [(W0, as0, ad0, b0, 2), (W1, as1, ad1, b1, 2),
              (W2, as2, ad2, b2, 2), (W3, as3, ad3, b3, 2),
              (W4, as4, ad4, b4, 1)]

    nb = N // _BLK
    acc = den_r = None
    bias_prev = None
    att = None
    for li, (w, a_s, a_d, b, heads) in enumerate(layers):
        if li == 0:
            hp, als, ald = _project(X, None, None, w, a_s, a_d, heads)
        else:
            hp, als, ald = _project(acc, den_r, bias_prev, w, a_s, a_d,
                                    heads)
        als_t = als.transpose(1, 0, 2).reshape(2, N)
        ald_t = ald.transpose(1, 0, 2).reshape(2, N)
        accs = []
        dens = []
        for hd in range(heads):
            alh = jnp.pad(als_t[hd], (0, NP - N))
            adh = jnp.pad(ald_t[hd], (0, NP - N))
            als2 = jnp.concatenate([alh, alh])
            ald2 = jnp.concatenate([adh, adh])
            hp2 = hp[2 * hd:2 * hd + 2].reshape(2 * N, DP)
            ek = _edge_kernel(li == 4)
            outs = ek(src1, dst1, als2, ald2, hp2)
            if li == 4:
                acc_h, den_h, attp = outs
                att = attp[:E2][:, None]
            else:
                acc_h, den_h = outs
            accs.append(acc_h)
            dens.append(den_h.reshape(80 * 128)[:N])
        acc = jnp.concatenate(accs, axis=0)
        if heads == 1:
            dens = [dens[0], dens[0]]
        den_r = (jnp.stack(dens).reshape(2, nb, _BLK).transpose(1, 0, 2))
        bias_prev = b

    x = _finalize(acc, den_r, b4)
    return x, att
